# baseline (speedup 1.0000x reference)
import functools

import jax
import jax.numpy as jnp
from jax import lax
from jax.experimental import pallas as pl
from jax.experimental.pallas import tpu as pltpu
from jax.experimental.pallas import tpu_sc as plsc

_DIM = 1024
_ROWS = 8192
_NC, _NS = 2, 16
_NW = _NC * _NS
_ROWS_PER_W = _ROWS // _NW
_CHUNK = 8
_NCHUNK = _ROWS_PER_W // _CHUNK
_NBUF = 14
_G = 7


@functools.partial(
    pl.kernel,
    mesh=plsc.VectorSubcoreMesh(core_axis_name="c", subcore_axis_name="s"),
    out_type=jax.ShapeDtypeStruct((_ROWS, _DIM), jnp.float32),
    scratch_types=(
        [pltpu.VMEM((_CHUNK, _DIM), jnp.float32) for _ in range(_NBUF)]
        + [pltpu.SemaphoreType.DMA for _ in range(2 * _NBUF)]
    ),
)
def _sc_copy(emb_hbm, out_hbm, *scratch):
    bufs = scratch[:_NBUF]
    gsems = scratch[_NBUF:2 * _NBUF]
    ssems = scratch[2 * _NBUF:]
    wid = lax.axis_index("s") * _NC + lax.axis_index("c")
    base = wid * _ROWS_PER_W

    def gather(i):
        b = i % _NBUF
        return pltpu.make_async_copy(
            emb_hbm.at[pl.ds(base + i * _CHUNK, _CHUNK)], bufs[b], gsems[b])

    def scatter(i):
        b = i % _NBUF
        return pltpu.make_async_copy(
            bufs[b], out_hbm.at[pl.ds(base + i * _CHUNK, _CHUNK)], ssems[b])

    for i in range(_G):
        gather(i).start()
    for i in range(_NCHUNK):
        gather(i).wait()
        scatter(i).start()
        j = i + _G
        if j < _NCHUNK:
            if j - _NBUF >= 0:
                scatter(j - _NBUF).wait()
            gather(j).start()
    for i in range(_NCHUNK - _NBUF, _NCHUNK):
        scatter(i).wait()


def kernel(x, emb_weight):
    del x
    return _sc_copy(emb_weight)[None, :, :]


# speedup vs baseline: 1.0464x; 1.0464x over previous
import functools

import jax
import jax.numpy as jnp
from jax import lax
from jax.experimental import pallas as pl
from jax.experimental.pallas import tpu as pltpu
from jax.experimental.pallas import tpu_sc as plsc

_DIM = 1024
_ROWS = 8192
_NC, _NS = 2, 16
_NW = _NC * _NS
_ROWS_PER_W = _ROWS // _NW
_CHUNK = 16
_NCHUNK = _ROWS_PER_W // _CHUNK
_NBUF = 7
_G = 5


@functools.partial(
    pl.kernel,
    mesh=plsc.VectorSubcoreMesh(core_axis_name="c", subcore_axis_name="s"),
    out_type=jax.ShapeDtypeStruct((_ROWS, _DIM), jnp.float32),
    scratch_types=(
        [pltpu.VMEM((_CHUNK, _DIM), jnp.float32) for _ in range(_NBUF)]
        + [pltpu.SemaphoreType.DMA for _ in range(2 * _NBUF)]
    ),
)
def _sc_copy(emb_hbm, out_hbm, *scratch):
    bufs = scratch[:_NBUF]
    gsems = scratch[_NBUF:2 * _NBUF]
    ssems = scratch[2 * _NBUF:]
    wid = lax.axis_index("s") * _NC + lax.axis_index("c")
    base = wid * _ROWS_PER_W

    def gather(i):
        b = i % _NBUF
        return pltpu.make_async_copy(
            emb_hbm.at[pl.ds(base + i * _CHUNK, _CHUNK)], bufs[b], gsems[b])

    def scatter(i):
        b = i % _NBUF
        return pltpu.make_async_copy(
            bufs[b], out_hbm.at[pl.ds(base + i * _CHUNK, _CHUNK)], ssems[b])

    for i in range(_G):
        gather(i).start()
    for i in range(_NCHUNK):
        gather(i).wait()
        scatter(i).start()
        j = i + _G
        if j < _NCHUNK:
            if j - _NBUF >= 0:
                scatter(j - _NBUF).wait()
            gather(j).start()
    for i in range(_NCHUNK - _NBUF, _NCHUNK):
        scatter(i).wait()


def kernel(x, emb_weight):
    del x
    return _sc_copy(emb_weight)[None, :, :]
